# baseline (speedup 1.0000x reference)
import functools

import jax
import jax.numpy as jnp
from jax import lax
from jax.experimental import pallas as pl
from jax.experimental.pallas import tpu as pltpu
from jax.experimental.pallas import tpu_sc as plsc

N = 10000
E = 320000
B = 64
D_IN = 78
D = 128
L = 3

SCALE = 1.0 / (D ** 0.5)

NP = 10240
NC = 2
NS = 16
NW = NC * NS
EPW = E // NW
CH1 = 2000
CB1 = 400
NCH1 = EPW // CH1
NBPC1 = CH1 // CB1
CH2 = 2000
CB2 = 200
NCH2 = EPW // CH2
NBPC2 = CH2 // CB2
RPT = NP // NS
FCH = 80

RB = 128
NB = NP // RB
OH = NP // RB

_SC_PARAMS = pltpu.CompilerParams(needs_layout_passes=False)
_MESH = plsc.VectorSubcoreMesh(core_axis_name="c", subcore_axis_name="s")


def _qk_vs(p, h, wq, bq, wk, bk, wv, bv, ws, bs):
    wa = jnp.where(p == 0, wq, wk)
    ba = jnp.where(p == 0, bq, bk)
    sc = jnp.where(p == 0, SCALE, 1.0)
    wb = jnp.where(p == 0, wv, ws)
    bb = jnp.where(p == 0, bv, bs)
    qk = (jnp.dot(h, wa, preferred_element_type=jnp.float32) + ba) * sc
    vs = jnp.dot(h, wb, preferred_element_type=jnp.float32) + bb
    return qk, vs


def _tc_first_body(x_ref, wp_ref, bp_ref, wq_ref, bq_ref, wk_ref, bk_ref,
                   wv_ref, bv_ref, ws_ref, bs_ref, qk_ref, vs_ref):
    p = pl.program_id(0)
    h = jnp.dot(x_ref[...], wp_ref[...],
                preferred_element_type=jnp.float32) + bp_ref[...]
    qk, vs = _qk_vs(p, h, wq_ref[...], bq_ref[...], wk_ref[...], bk_ref[...],
                    wv_ref[...], bv_ref[...], ws_ref[...], bs_ref[...])
    qk_ref[...] = qk
    vs_ref[...] = vs


def _tc_first(x, wp, bp, wq, bq, wk, bk, wv, bv, ws, bs):
    full = lambda shape: pl.BlockSpec(shape, lambda p, j: (0, 0))
    return pl.pallas_call(
        _tc_first_body,
        grid=(2, NB),
        in_specs=[
            pl.BlockSpec((RB, D_IN), lambda p, j: (j, 0)),
            full((D_IN, D)), full((1, D)),
            full((D, D)), full((1, D)),
            full((D, D)), full((1, D)),
            full((D, D)), full((1, D)),
            full((D, D)), full((1, D)),
        ],
        out_specs=[pl.BlockSpec((RB, D), lambda p, j: (p * OH + j, 0))] * 2,
        out_shape=[jax.ShapeDtypeStruct((2 * NP, D), jnp.float32)] * 2,
    )(x, wp, bp, wq, bq, wk, bk, wv, bv, ws, bs)


def _leaky_h(o0, o1, dp, sp):
    den = jnp.maximum(jnp.sum(dp, axis=0), 1e-30)
    h = (o0 + o1) / den[:, None] + sp
    return jnp.where(h >= 0, h, 0.01 * h)


def _tc_mid_body(o0_ref, o1_ref, dp_ref, sp_ref, wq_ref, bq_ref, wk_ref,
                 bk_ref, wv_ref, bv_ref, ws_ref, bs_ref, qk_ref, vs_ref):
    p = pl.program_id(0)
    h = _leaky_h(o0_ref[...], o1_ref[...], dp_ref[...], sp_ref[...])
    qk, vs = _qk_vs(p, h, wq_ref[...], bq_ref[...], wk_ref[...], bk_ref[...],
                    wv_ref[...], bv_ref[...], ws_ref[...], bs_ref[...])
    qk_ref[...] = qk
    vs_ref[...] = vs


def _tc_mid(outp, dpart, vs_prev, wq, bq, wk, bk, wv, bv, ws, bs):
    full = lambda shape: pl.BlockSpec(shape, lambda p, j: (0, 0))
    return pl.pallas_call(
        _tc_mid_body,
        grid=(2, NB),
        in_specs=[
            pl.BlockSpec((RB, D), lambda p, j: (j, 0)),
            pl.BlockSpec((RB, D), lambda p, j: (j + OH, 0)),
            pl.BlockSpec((NW, RB), lambda p, j: (0, j)),
            pl.BlockSpec((RB, D), lambda p, j: (j + OH, 0)),
            full((D, D)), full((1, D)),
            full((D, D)), full((1, D)),
            full((D, D)), full((1, D)),
            full((D, D)), full((1, D)),
        ],
        out_specs=[pl.BlockSpec((RB, D), lambda p, j: (p * OH + j, 0))] * 2,
        out_shape=[jax.ShapeDtypeStruct((2 * NP, D), jnp.float32)] * 2,
    )(outp, outp, dpart, vs_prev, wq, bq, wk, bk, wv, bv, ws, bs)


def _tc_pool_body(o0_ref, o1_ref, dp_ref, sp_ref, b_ref, out_ref,
                  acc_ref, cnt_ref):
    j = pl.program_id(0)
    h = _leaky_h(o0_ref[...], o1_ref[...], dp_ref[...], sp_ref[...])
    bvec = b_ref[0, 0, :]
    onehot = (bvec[:, None]
              == lax.broadcasted_iota(jnp.int32, (RB, B), 1)
              ).astype(jnp.float32)

    @pl.when(j == 0)
    def _():
        acc_ref[...] = jnp.zeros((B, D), jnp.float32)
        cnt_ref[...] = jnp.zeros((B, D), jnp.float32)

    acc_ref[...] += lax.dot_general(
        onehot, h, (((0,), (0,)), ((), ())),
        preferred_element_type=jnp.float32)
    cnt_ref[...] += jnp.broadcast_to(
        jnp.sum(onehot, axis=0)[:, None], (B, D))

    @pl.when(j == NB - 1)
    def _():
        out_ref[...] = acc_ref[...] / jnp.maximum(cnt_ref[...], 1.0)


def _tc_pool(outp, dpart, vs_prev, batch3):
    return pl.pallas_call(
        _tc_pool_body,
        grid=(NB,),
        in_specs=[
            pl.BlockSpec((RB, D), lambda j: (j, 0)),
            pl.BlockSpec((RB, D), lambda j: (j + OH, 0)),
            pl.BlockSpec((NW, RB), lambda j: (0, j)),
            pl.BlockSpec((RB, D), lambda j: (j + OH, 0)),
            pl.BlockSpec((1, 1, RB), lambda j: (j, 0, 0)),
        ],
        out_specs=pl.BlockSpec((B, D), lambda j: (0, 0)),
        out_shape=jax.ShapeDtypeStruct((B, D), jnp.float32),
        scratch_shapes=[
            pltpu.VMEM((B, D), jnp.float32),
            pltpu.VMEM((B, D), jnp.float32),
        ],
    )(outp, outp, dpart, vs_prev, batch3)


def _sc_alpha_body(qk_hbm, qkidx_hbm, dst_hbm, ex_hbm, dpart_hbm,
                   idx_v, dst_v, rows_v, ex_v, den_v, sem0):
    c = lax.axis_index("c")
    s = lax.axis_index("s")
    wid = s * NC + c
    zero16 = jnp.zeros((16,), jnp.float32)
    iota16 = lax.iota(jnp.int32, 16)

    def zrow(i, _):
        den_v[pl.ds(i * 16, 16)] = zero16
        return 0
    lax.fori_loop(0, NP // 16, zrow, 0)

    ebase = wid * EPW

    def chunk(ci, _):
        cbase = ebase + ci * CH1
        pltpu.sync_copy(qkidx_hbm.at[pl.ds(2 * cbase, 2 * CH1)], idx_v)
        pltpu.sync_copy(dst_hbm.at[pl.ds(cbase, CH1)], dst_v)

        def blk(b, _1):
            e0 = b * CB1
            pltpu.async_copy(
                qk_hbm.at[idx_v.at[pl.ds(2 * e0, 2 * CB1)]], rows_v,
                sem0).wait()

            def grp(g, _2):
                r0 = g * 16
                row16 = iota16 + r0
                dst16 = dst_v[pl.ds(e0 + r0, 16)]
                acc = zero16
                for dd in range(D):
                    col = jnp.full((16,), dd, jnp.int32)
                    qv = plsc.load_gather(rows_v, [row16, col])
                    kv = plsc.load_gather(rows_v, [row16 + CB1, col])
                    acc = acc + qv * kv
                exv = jnp.exp(acc)
                ex_v[pl.ds(e0 + r0, 16)] = exv
                plsc.addupdate_scatter(den_v, [dst16], exv)
                return 0
            lax.fori_loop(0, CB1 // 16, grp, 0)
            return 0
        lax.fori_loop(0, NBPC1, blk, 0)
        pltpu.sync_copy(ex_v, ex_hbm.at[pl.ds(cbase, CH1)])
        return 0
    lax.fori_loop(0, NCH1, chunk, 0)

    pltpu.sync_copy(den_v, dpart_hbm.at[wid])


def _sc_alpha(qk, qk_idx, dst):
    run = functools.partial(
        pl.kernel,
        out_type=(jax.ShapeDtypeStruct((E,), jnp.float32),
                  jax.ShapeDtypeStruct((NW, NP), jnp.float32)),
        mesh=_MESH,
        compiler_params=_SC_PARAMS,
        scratch_types=[
            pltpu.VMEM((2 * CH1,), jnp.int32),
            pltpu.VMEM((CH1,), jnp.int32),
            pltpu.VMEM((2 * CB1, D), jnp.float32),
            pltpu.VMEM((CH1,), jnp.float32),
            pltpu.VMEM((NP,), jnp.float32),
            pltpu.SemaphoreType.DMA,
        ],
    )(_sc_alpha_body)
    return run(qk, qk_idx, dst)


def _sc_agg_body(vs_hbm, ex_hbm, src_hbm, dst_hbm, zeros_hbm, outp_hbm,
                 src_v, dst_v, ex_v, vrows_v, spmem_out, sem0):
    c = lax.axis_index("c")
    s = lax.axis_index("s")
    wid = s * NC + c

    pltpu.sync_copy(zeros_hbm, spmem_out.at[pl.ds(s * RPT, RPT)])
    plsc.subcore_barrier()

    ebase = wid * EPW

    def chunk(ci, _):
        cbase = ebase + ci * CH2
        pltpu.sync_copy(src_hbm.at[pl.ds(cbase, CH2)], src_v)
        pltpu.sync_copy(ex_hbm.at[pl.ds(cbase, CH2)], ex_v)

        def blk(b, _1):
            e0 = b * CB2
            pltpu.sync_copy(dst_hbm.at[pl.ds(cbase + e0, CB2)], dst_v)
            pltpu.async_copy(
                vs_hbm.at[src_v.at[pl.ds(e0, CB2)]], vrows_v, sem0).wait()

            def scale(e5, _2):
                for u in range(5):
                    e = e5 * 5 + u
                    sp = plsc.load_gather(
                        ex_v, [jnp.full((16,), e0 + e, jnp.int32)])
                    for cc in range(D // 16):
                        vrows_v[e, pl.ds(cc * 16, 16)] = (
                            vrows_v[e, pl.ds(cc * 16, 16)] * sp)
                return 0
            lax.fori_loop(0, CB2 // 5, scale, 0)

            pltpu.sync_copy(vrows_v, spmem_out.at[dst_v], add=True)
            return 0
        lax.fori_loop(0, NBPC2, blk, 0)
        return 0
    lax.fori_loop(0, NCH2, chunk, 0)

    plsc.subcore_barrier()

    def flush(t, _):
        r0 = s * RPT + t * FCH
        pltpu.sync_copy(spmem_out.at[pl.ds(r0, FCH)],
                        vrows_v.at[pl.ds(0, FCH)])
        pltpu.sync_copy(vrows_v.at[pl.ds(0, FCH)],
                        outp_hbm.at[pl.ds(c * NP + r0, FCH)])
        return 0
    lax.fori_loop(0, RPT // FCH, flush, 0)


def _sc_agg(vs, ex, src, dst, zeros):
    run = functools.partial(
        pl.kernel,
        out_type=jax.ShapeDtypeStruct((NC * NP, D), jnp.float32),
        mesh=_MESH,
        compiler_params=_SC_PARAMS,
        scratch_types=[
            pltpu.VMEM((CH2,), jnp.int32),
            pltpu.VMEM((CB2,), jnp.int32),
            pltpu.VMEM((CH2,), jnp.float32),
            pltpu.VMEM((CB2, D), jnp.float32),
            pltpu.VMEM_SHARED((NP, D), jnp.float32),
            pltpu.SemaphoreType.DMA,
        ],
    )(_sc_agg_body)
    return run(vs, ex, src, dst, zeros)


def kernel(x, edge_index, batch, Wp, bp, Wq, bq, Wk, bk, Wv, bv, Ws, bs):
    src = edge_index[0]
    dst = edge_index[1]
    qk_idx = jnp.concatenate(
        [dst.reshape(-1, CB1), src.reshape(-1, CB1) + NP], axis=1).reshape(-1)
    xp = jnp.pad(x, ((0, NP - N), (0, 0)))
    batch_p = jnp.concatenate(
        [batch, jnp.full((NP - N,), B, jnp.int32)]).reshape(NB, 1, RB)
    zeros = jnp.zeros((RPT, D), jnp.float32)
    b2 = lambda b: b.reshape(1, D)

    qk, vs = _tc_first(xp, Wp, b2(bp),
                       Wq[0], b2(bq[0]), Wk[0], b2(bk[0]),
                       Wv[0], b2(bv[0]), Ws[0], b2(bs[0]))
    for i in range(L):
        ex, dpart = _sc_alpha(qk, qk_idx, dst)
        outp = _sc_agg(vs, ex, src, dst, zeros)
        if i < L - 1:
            qk, vs = _tc_mid(outp, dpart, vs,
                             Wq[i + 1], b2(bq[i + 1]),
                             Wk[i + 1], b2(bk[i + 1]),
                             Wv[i + 1], b2(bv[i + 1]),
                             Ws[i + 1], b2(bs[i + 1]))
    return _tc_pool(outp, dpart, vs, batch_p)


# speedup vs baseline: 2.3832x; 2.3832x over previous
import functools

import jax
import jax.numpy as jnp
from jax import lax
from jax.experimental import pallas as pl
from jax.experimental.pallas import tpu as pltpu
from jax.experimental.pallas import tpu_sc as plsc

N = 10000
E = 320000
B = 64
D_IN = 78
D = 128
L = 3

SCALE = 1.0 / (D ** 0.5)

NP = 10240
NC = 2
NS = 16
NW = NC * NS
EPW = E // NW
CH1 = 2000
CB1 = 400
NCH1 = EPW // CH1
NBPC1 = CH1 // CB1
CH2 = 2000
CB2 = 200
NCH2 = EPW // CH2
NBPC2 = CH2 // CB2
RPT = NP // NS
FCH = 80

RB = 128
NB = NP // RB
OH = NP // RB

_SC_PARAMS = pltpu.CompilerParams(needs_layout_passes=False)
_MESH = plsc.VectorSubcoreMesh(core_axis_name="c", subcore_axis_name="s")


def _qk_vs(p, h, wq, bq, wk, bk, wv, bv, ws, bs):
    wa = jnp.where(p == 0, wq, wk)
    ba = jnp.where(p == 0, bq, bk)
    sc = jnp.where(p == 0, SCALE, 1.0)
    wb = jnp.where(p == 0, wv, ws)
    bb = jnp.where(p == 0, bv, bs)
    qk = (jnp.dot(h, wa, preferred_element_type=jnp.float32) + ba) * sc
    vs = jnp.dot(h, wb, preferred_element_type=jnp.float32) + bb
    return qk, vs


def _tc_first_body(x_ref, wp_ref, bp_ref, wq_ref, bq_ref, wk_ref, bk_ref,
                   wv_ref, bv_ref, ws_ref, bs_ref, qk_ref, vs_ref):
    p = pl.program_id(0)
    h = jnp.dot(x_ref[...], wp_ref[...],
                preferred_element_type=jnp.float32) + bp_ref[...]
    qk, vs = _qk_vs(p, h, wq_ref[...], bq_ref[...], wk_ref[...], bk_ref[...],
                    wv_ref[...], bv_ref[...], ws_ref[...], bs_ref[...])
    qk_ref[...] = qk
    vs_ref[...] = vs


def _tc_first(x, wp, bp, wq, bq, wk, bk, wv, bv, ws, bs):
    full = lambda shape: pl.BlockSpec(shape, lambda p, j: (0, 0))
    return pl.pallas_call(
        _tc_first_body,
        grid=(2, NB),
        in_specs=[
            pl.BlockSpec((RB, D_IN), lambda p, j: (j, 0)),
            full((D_IN, D)), full((1, D)),
            full((D, D)), full((1, D)),
            full((D, D)), full((1, D)),
            full((D, D)), full((1, D)),
            full((D, D)), full((1, D)),
        ],
        out_specs=[pl.BlockSpec((RB, D), lambda p, j: (p * OH + j, 0))] * 2,
        out_shape=[jax.ShapeDtypeStruct((2 * NP, D), jnp.float32)] * 2,
    )(x, wp, bp, wq, bq, wk, bk, wv, bv, ws, bs)


def _leaky_h(o0, o1, dp, sp):
    den = jnp.maximum(jnp.sum(dp, axis=0), 1e-30)
    h = (o0 + o1) / den[:, None] + sp
    return jnp.where(h >= 0, h, 0.01 * h)


def _tc_mid_body(o0_ref, o1_ref, dp_ref, sp_ref, wq_ref, bq_ref, wk_ref,
                 bk_ref, wv_ref, bv_ref, ws_ref, bs_ref, qk_ref, vs_ref):
    p = pl.program_id(0)
    h = _leaky_h(o0_ref[...], o1_ref[...], dp_ref[...], sp_ref[...])
    qk, vs = _qk_vs(p, h, wq_ref[...], bq_ref[...], wk_ref[...], bk_ref[...],
                    wv_ref[...], bv_ref[...], ws_ref[...], bs_ref[...])
    qk_ref[...] = qk
    vs_ref[...] = vs


def _tc_mid(outp, dpart, vs_prev, wq, bq, wk, bk, wv, bv, ws, bs):
    full = lambda shape: pl.BlockSpec(shape, lambda p, j: (0, 0))
    return pl.pallas_call(
        _tc_mid_body,
        grid=(2, NB),
        in_specs=[
            pl.BlockSpec((RB, D), lambda p, j: (j, 0)),
            pl.BlockSpec((RB, D), lambda p, j: (j + OH, 0)),
            pl.BlockSpec((NW, RB), lambda p, j: (0, j)),
            pl.BlockSpec((RB, D), lambda p, j: (j + OH, 0)),
            full((D, D)), full((1, D)),
            full((D, D)), full((1, D)),
            full((D, D)), full((1, D)),
            full((D, D)), full((1, D)),
        ],
        out_specs=[pl.BlockSpec((RB, D), lambda p, j: (p * OH + j, 0))] * 2,
        out_shape=[jax.ShapeDtypeStruct((2 * NP, D), jnp.float32)] * 2,
    )(outp, outp, dpart, vs_prev, wq, bq, wk, bk, wv, bv, ws, bs)


def _tc_pool_body(o0_ref, o1_ref, dp_ref, sp_ref, b_ref, out_ref,
                  acc_ref, cnt_ref):
    j = pl.program_id(0)
    h = _leaky_h(o0_ref[...], o1_ref[...], dp_ref[...], sp_ref[...])
    bvec = b_ref[0, 0, :]
    onehot = (bvec[:, None]
              == lax.broadcasted_iota(jnp.int32, (RB, B), 1)
              ).astype(jnp.float32)

    @pl.when(j == 0)
    def _():
        acc_ref[...] = jnp.zeros((B, D), jnp.float32)
        cnt_ref[...] = jnp.zeros((B, D), jnp.float32)

    acc_ref[...] += lax.dot_general(
        onehot, h, (((0,), (0,)), ((), ())),
        preferred_element_type=jnp.float32)
    cnt_ref[...] += jnp.broadcast_to(
        jnp.sum(onehot, axis=0)[:, None], (B, D))

    @pl.when(j == NB - 1)
    def _():
        out_ref[...] = acc_ref[...] / jnp.maximum(cnt_ref[...], 1.0)


def _tc_pool(outp, dpart, vs_prev, batch3):
    return pl.pallas_call(
        _tc_pool_body,
        grid=(NB,),
        in_specs=[
            pl.BlockSpec((RB, D), lambda j: (j, 0)),
            pl.BlockSpec((RB, D), lambda j: (j + OH, 0)),
            pl.BlockSpec((NW, RB), lambda j: (0, j)),
            pl.BlockSpec((RB, D), lambda j: (j + OH, 0)),
            pl.BlockSpec((1, 1, RB), lambda j: (j, 0, 0)),
        ],
        out_specs=pl.BlockSpec((B, D), lambda j: (0, 0)),
        out_shape=jax.ShapeDtypeStruct((B, D), jnp.float32),
        scratch_shapes=[
            pltpu.VMEM((B, D), jnp.float32),
            pltpu.VMEM((B, D), jnp.float32),
        ],
    )(outp, outp, dpart, vs_prev, batch3)


def _sc_alpha_body(qk_hbm, qkidx_hbm, dst_hbm, ex_hbm, dpart_hbm,
                   idx_v, dst_v, rows_v, ex_v, den_v, sem0):
    c = lax.axis_index("c")
    s = lax.axis_index("s")
    wid = s * NC + c
    zero16 = jnp.zeros((16,), jnp.float32)
    iota16 = lax.iota(jnp.int32, 16)

    def zrow(i, _):
        den_v[pl.ds(i * 16, 16)] = zero16
        return 0
    lax.fori_loop(0, NP // 16, zrow, 0)

    ebase = wid * EPW

    def chunk(ci, _):
        cbase = ebase + ci * CH1
        pltpu.sync_copy(qkidx_hbm.at[pl.ds(2 * cbase, 2 * CH1)], idx_v)
        pltpu.sync_copy(dst_hbm.at[pl.ds(cbase, CH1)], dst_v)

        def blk(b, _1):
            e0 = b * CB1
            pltpu.async_copy(
                qk_hbm.at[idx_v.at[pl.ds(2 * e0, 2 * CB1)]], rows_v,
                sem0).wait()

            def grp(g, _2):
                r0 = g * 16
                row16 = iota16 + r0
                dst16 = dst_v[pl.ds(e0 + r0, 16)]
                acc = zero16
                for dd in range(D):
                    col = jnp.bitwise_and(iota16 + dd, D - 1)
                    qv = plsc.load_gather(rows_v, [row16, col])
                    kv = plsc.load_gather(rows_v, [row16 + CB1, col])
                    acc = acc + qv * kv
                exv = jnp.exp(acc)
                ex_v[pl.ds(e0 + r0, 16)] = exv
                plsc.addupdate_scatter(den_v, [dst16], exv)
                return 0
            lax.fori_loop(0, CB1 // 16, grp, 0)
            return 0
        lax.fori_loop(0, NBPC1, blk, 0)
        pltpu.sync_copy(ex_v, ex_hbm.at[pl.ds(cbase, CH1)])
        return 0
    lax.fori_loop(0, NCH1, chunk, 0)

    pltpu.sync_copy(den_v, dpart_hbm.at[wid])


def _sc_alpha(qk, qk_idx, dst):
    run = functools.partial(
        pl.kernel,
        out_type=(jax.ShapeDtypeStruct((E,), jnp.float32),
                  jax.ShapeDtypeStruct((NW, NP), jnp.float32)),
        mesh=_MESH,
        compiler_params=_SC_PARAMS,
        scratch_types=[
            pltpu.VMEM((2 * CH1,), jnp.int32),
            pltpu.VMEM((CH1,), jnp.int32),
            pltpu.VMEM((2 * CB1, D), jnp.float32),
            pltpu.VMEM((CH1,), jnp.float32),
            pltpu.VMEM((NP,), jnp.float32),
            pltpu.SemaphoreType.DMA,
        ],
    )(_sc_alpha_body)
    return run(qk, qk_idx, dst)


def _sc_agg_body(vs_hbm, ex_hbm, src_hbm, dst_hbm, zeros_hbm, outp_hbm,
                 src_v, dst_v, ex_v, vrows_v, spmem_out, sem0):
    c = lax.axis_index("c")
    s = lax.axis_index("s")
    wid = s * NC + c

    pltpu.sync_copy(zeros_hbm, spmem_out.at[pl.ds(s * RPT, RPT)])
    plsc.subcore_barrier()

    ebase = wid * EPW

    def chunk(ci, _):
        cbase = ebase + ci * CH2
        pltpu.sync_copy(src_hbm.at[pl.ds(cbase, CH2)], src_v)
        pltpu.sync_copy(ex_hbm.at[pl.ds(cbase, CH2)], ex_v)

        def blk(b, _1):
            e0 = b * CB2
            pltpu.sync_copy(dst_hbm.at[pl.ds(cbase + e0, CB2)], dst_v)
            pltpu.async_copy(
                vs_hbm.at[src_v.at[pl.ds(e0, CB2)]], vrows_v, sem0).wait()

            def scale(e5, _2):
                for u in range(5):
                    e = e5 * 5 + u
                    sp = plsc.load_gather(
                        ex_v, [jnp.full((16,), e0 + e, jnp.int32)])
                    for cc in range(D // 16):
                        vrows_v[e, pl.ds(cc * 16, 16)] = (
                            vrows_v[e, pl.ds(cc * 16, 16)] * sp)
                return 0
            lax.fori_loop(0, CB2 // 5, scale, 0)

            pltpu.sync_copy(vrows_v, spmem_out.at[dst_v], add=True)
            return 0
        lax.fori_loop(0, NBPC2, blk, 0)
        return 0
    lax.fori_loop(0, NCH2, chunk, 0)

    plsc.subcore_barrier()

    def flush(t, _):
        r0 = s * RPT + t * FCH
        pltpu.sync_copy(spmem_out.at[pl.ds(r0, FCH)],
                        vrows_v.at[pl.ds(0, FCH)])
        pltpu.sync_copy(vrows_v.at[pl.ds(0, FCH)],
                        outp_hbm.at[pl.ds(c * NP + r0, FCH)])
        return 0
    lax.fori_loop(0, RPT // FCH, flush, 0)


def _sc_agg(vs, ex, src, dst, zeros):
    run = functools.partial(
        pl.kernel,
        out_type=jax.ShapeDtypeStruct((NC * NP, D), jnp.float32),
        mesh=_MESH,
        compiler_params=_SC_PARAMS,
        scratch_types=[
            pltpu.VMEM((CH2,), jnp.int32),
            pltpu.VMEM((CB2,), jnp.int32),
            pltpu.VMEM((CH2,), jnp.float32),
            pltpu.VMEM((CB2, D), jnp.float32),
            pltpu.VMEM_SHARED((NP, D), jnp.float32),
            pltpu.SemaphoreType.DMA,
        ],
    )(_sc_agg_body)
    return run(vs, ex, src, dst, zeros)


def kernel(x, edge_index, batch, Wp, bp, Wq, bq, Wk, bk, Wv, bv, Ws, bs):
    src = edge_index[0]
    dst = edge_index[1]
    qk_idx = jnp.concatenate(
        [dst.reshape(-1, CB1), src.reshape(-1, CB1) + NP], axis=1).reshape(-1)
    xp = jnp.pad(x, ((0, NP - N), (0, 0)))
    batch_p = jnp.concatenate(
        [batch, jnp.full((NP - N,), B, jnp.int32)]).reshape(NB, 1, RB)
    zeros = jnp.zeros((RPT, D), jnp.float32)
    b2 = lambda b: b.reshape(1, D)

    qk, vs = _tc_first(xp, Wp, b2(bp),
                       Wq[0], b2(bq[0]), Wk[0], b2(bk[0]),
                       Wv[0], b2(bv[0]), Ws[0], b2(bs[0]))
    for i in range(L):
        ex, dpart = _sc_alpha(qk, qk_idx, dst)
        outp = _sc_agg(vs, ex, src, dst, zeros)
        if i < L - 1:
            qk, vs = _tc_mid(outp, dpart, vs,
                             Wq[i + 1], b2(bq[i + 1]),
                             Wk[i + 1], b2(bk[i + 1]),
                             Wv[i + 1], b2(bv[i + 1]),
                             Ws[i + 1], b2(bs[i + 1]))
    return _tc_pool(outp, dpart, vs, batch_p)
